# baseline (speedup 1.0000x reference)
import jax
import jax.numpy as jnp
from jax import lax
from jax.experimental import pallas as pl
from jax.experimental.pallas import tpu as pltpu
from jax.experimental.pallas import tpu_sc as plsc

_B = 16384
_L = 200
_PAD = 1000
_TAB = 1008
_NC = 2
_NS = 16
_NW = _NC * _NS
_ROWS = _B // _NW
_LN = 16


_CH = 128
_NCHUNK = _ROWS // _CH


def _noisy_or_body(rules_hbm, w_hbm, out_hbm, r0, r1, tab_v, out_v, s0, s1):
    wid = lax.axis_index("s") * _NC + lax.axis_index("c")
    base = wid * _ROWS
    bufs, sems = [r0, r1], [s0, s1]

    copies = [
        pltpu.make_async_copy(
            rules_hbm.at[pl.ds(base + c * _CH, _CH), :], bufs[c % 2], sems[c % 2]
        )
        for c in range(_NCHUNK)
    ]
    copies[0].start()

    pltpu.sync_copy(w_hbm, tab_v)

    def tbuild(j, c):
        w = tab_v[pl.ds(j * _LN, _LN)]
        p = 1.0 / (1.0 + jnp.exp(w))
        gidx = j * _LN + lax.broadcasted_iota(jnp.int32, (_LN,), 0)
        tab_v[pl.ds(j * _LN, _LN)] = jnp.where(gidx == _PAD, 1.0, p)
        return c

    lax.fori_loop(0, _TAB // _LN, tbuild, 0)

    lanes = lax.broadcasted_iota(jnp.int32, (_LN,), 0)

    for c in range(_NCHUNK):
        if c + 1 < _NCHUNK:
            copies[c + 1].start()
        copies[c].wait()
        buf = bufs[c % 2]

        def group(g, _, c=c, buf=buf):
            rows = g * _LN + lanes

            def ibody(t, accs, buf=buf, rows=rows):
                new = list(accs)
                for u in range(8):
                    cols = jnp.broadcast_to(t * 8 + u, (_LN,))
                    ids = plsc.load_gather(buf, [rows, cols])
                    vals = plsc.load_gather(tab_v, [ids])
                    new[u] = new[u] * vals
                return tuple(new)

            ones = jnp.full((_LN,), 1.0, jnp.float32)
            accs = lax.fori_loop(0, _L // 8, ibody, (ones,) * 8)
            acc = ((accs[0] * accs[1]) * (accs[2] * accs[3])) * (
                (accs[4] * accs[5]) * (accs[6] * accs[7])
            )
            out_v[pl.ds(c * _CH + g * _LN, _LN)] = jnp.clip(
                1.0 - acc, 1e-4, 1.0 - 1e-5
            )
            return _

        lax.fori_loop(0, _CH // _LN, group, 0)

    pltpu.sync_copy(out_v, out_hbm.at[pl.ds(base, _ROWS)])


def kernel(rules, W):
    wp = jnp.concatenate(
        [W.reshape(-1).astype(jnp.float32),
         jnp.zeros((_TAB - _PAD - 1,), jnp.float32)]
    )
    f = pl.kernel(
        _noisy_or_body,
        mesh=plsc.VectorSubcoreMesh(core_axis_name="c", subcore_axis_name="s"),
        compiler_params=pltpu.CompilerParams(
            needs_layout_passes=False, use_tc_tiling_on_sc=True
        ),
        out_type=jax.ShapeDtypeStruct((_B,), jnp.float32),
        scratch_types=[
            pltpu.VMEM((_CH, _L), jnp.int32),
            pltpu.VMEM((_CH, _L), jnp.int32),
            pltpu.VMEM((_TAB,), jnp.float32),
            pltpu.VMEM((_ROWS,), jnp.float32),
            pltpu.SemaphoreType.DMA,
            pltpu.SemaphoreType.DMA,
        ],
    )
    return f(rules.astype(jnp.int32), wp).reshape(_B, 1)


# speedup vs baseline: 1.6028x; 1.6028x over previous
import jax
import jax.numpy as jnp
from jax import lax
from jax.experimental import pallas as pl
from jax.experimental.pallas import tpu as pltpu
from jax.experimental.pallas import tpu_sc as plsc

_B = 16384
_L = 200
_PAD = 1000
_TAB = 1008
_NC = 2
_NS = 16
_NW = _NC * _NS
_ROWS = _B // _NW
_LN = 16
_HALF = _ROWS // 2


def _noisy_or_body(rt_hbm, w_hbm, out_hbm, buf, tab_v, out_v, s0, s1):
    wid = lax.axis_index("s") * _NC + lax.axis_index("c")
    base = wid * _ROWS

    cp0 = pltpu.make_async_copy(
        rt_hbm.at[:, pl.ds(base, _HALF)], buf.at[:, pl.ds(0, _HALF)], s0
    )
    cp1 = pltpu.make_async_copy(
        rt_hbm.at[:, pl.ds(base + _HALF, _HALF)],
        buf.at[:, pl.ds(_HALF, _HALF)],
        s1,
    )
    cp0.start()
    cp1.start()

    pltpu.sync_copy(w_hbm, tab_v)

    def tbuild(j, c):
        w = tab_v[pl.ds(j * _LN, _LN)]
        p = 1.0 / (1.0 + jnp.exp(w))
        gidx = j * _LN + lax.broadcasted_iota(jnp.int32, (_LN,), 0)
        tab_v[pl.ds(j * _LN, _LN)] = jnp.where(gidx == _PAD, 1.0, p)
        return c

    lax.fori_loop(0, _TAB // _LN, tbuild, 0)

    def group(g, c):
        col0 = g * _LN
        accs = [None] * 8
        for i in range(_L):
            ids = buf[i, pl.ds(col0, _LN)]
            vals = plsc.load_gather(tab_v, [ids])
            k = i % 8
            accs[k] = vals if accs[k] is None else accs[k] * vals
        acc = ((accs[0] * accs[1]) * (accs[2] * accs[3])) * (
            (accs[4] * accs[5]) * (accs[6] * accs[7])
        )
        out_v[pl.ds(col0, _LN)] = jnp.clip(1.0 - acc, 1e-4, 1.0 - 1e-5)
        return c

    cp0.wait()
    lax.fori_loop(0, _HALF // _LN, group, 0)
    cp1.wait()
    lax.fori_loop(_HALF // _LN, _ROWS // _LN, group, 0)

    pltpu.sync_copy(out_v, out_hbm.at[pl.ds(base, _ROWS)])


def kernel(rules, W):
    wp = jnp.concatenate(
        [W.reshape(-1).astype(jnp.float32),
         jnp.zeros((_TAB - _PAD - 1,), jnp.float32)]
    )
    f = pl.kernel(
        _noisy_or_body,
        mesh=plsc.VectorSubcoreMesh(core_axis_name="c", subcore_axis_name="s"),
        compiler_params=pltpu.CompilerParams(
            needs_layout_passes=False, use_tc_tiling_on_sc=True
        ),
        out_type=jax.ShapeDtypeStruct((_B,), jnp.float32),
        scratch_types=[
            pltpu.VMEM((_L, _ROWS), jnp.int32),
            pltpu.VMEM((_TAB,), jnp.float32),
            pltpu.VMEM((_ROWS,), jnp.float32),
            pltpu.SemaphoreType.DMA,
            pltpu.SemaphoreType.DMA,
        ],
    )
    return f(rules.astype(jnp.int32).T, wp).reshape(_B, 1)
